# baseline (device time: 168642 ns/iter reference)
import jax
import jax.numpy as jnp
from jax import lax
from jax.experimental import pallas as pl
from jax.experimental.pallas import tpu as pltpu

M = 2048
N = 2048
K = 8192
NX, NY, NZ = 2, 2, 4
N_SLABS = NX * NY
SLAB = M // N_SLABS
CHUNK = SLAB // NZ

BN = 512
BK = 2048

_CompilerParams = getattr(pltpu, "CompilerParams", None) or pltpu.TPUCompilerParams


def _gemm_body(s_ref, dy_ref, w_ref, out_ref, acc_ref):
    k = pl.program_id(1)

    @pl.when(k == 0)
    def _():
        acc_ref[...] = jnp.zeros_like(acc_ref)

    acc_ref[...] += lax.dot_general(
        dy_ref[...],
        w_ref[...],
        dimension_numbers=(((1,), (1,)), ((), ())),
        preferred_element_type=jnp.float32,
    )

    @pl.when(k == pl.num_programs(1) - 1)
    def _():
        out_ref[...] = acc_ref[...]


def _local_gemm(dy, w, s):
    grid_spec = pltpu.PrefetchScalarGridSpec(
        num_scalar_prefetch=1,
        grid=(N // BN, K // BK),
        in_specs=[
            pl.BlockSpec((SLAB, BK), lambda n, k, s_ref: (s_ref[0], k)),
            pl.BlockSpec((BN, BK), lambda n, k, s_ref: (n, k)),
        ],
        out_specs=pl.BlockSpec((SLAB, BN), lambda n, k, s_ref: (0, n)),
        scratch_shapes=[pltpu.VMEM((SLAB, BN), jnp.float32)],
    )
    return pl.pallas_call(
        _gemm_body,
        grid_spec=grid_spec,
        out_shape=jax.ShapeDtypeStruct((SLAB, N), jnp.float32),
        compiler_params=_CompilerParams(
            dimension_semantics=("arbitrary", "arbitrary"),
        ),
    )(jnp.asarray(s, jnp.int32).reshape(1), dy, w)


def _ring_pos(x, y):
    return 2 * x + jnp.bitwise_xor(x, y)


def _coords_of_ring_pos(p):
    x = p // 2
    y = ((p + 1) // 2) % 2
    return x, y


def _slab_of_ring_pos(p):
    x, y = _coords_of_ring_pos(p)
    return 2 * x + y


def _comm_body(partial_ref, out_ref, z_comm, z_ss, z_rs, xy_ss, xy_rs):
    mx = lax.axis_index("x")
    my = lax.axis_index("y")
    mz = lax.axis_index("z")
    s = 2 * mx + my
    r = _ring_pos(mx, my)
    base = s * SLAB

    z_right = (mx, my, (mz + 1) % NZ)
    z_left = (mx, my, (mz - 1) % NZ)
    xr, yr = _coords_of_ring_pos((r + 1) % N_SLABS)
    xl, yl = _coords_of_ring_pos((r - 1) % N_SLABS)
    xy_right = (xr, yr, mz)
    xy_left = (xl, yl, mz)

    bsem = pltpu.get_barrier_semaphore()
    for nbr in (z_left, z_right, xy_left, xy_right):
        pl.semaphore_signal(
            bsem, inc=1, device_id=nbr, device_id_type=pl.DeviceIdType.MESH
        )
    pl.semaphore_wait(bsem, 4)

    out_ref[pl.ds(base, SLAB), :] = partial_ref[...]

    for t in range(NZ - 1):
        c_send = (mz - t) % NZ
        c_recv = (mz - t - 1) % NZ
        rdma = pltpu.make_async_remote_copy(
            src_ref=out_ref.at[pl.ds(base + c_send * CHUNK, CHUNK), :],
            dst_ref=z_comm.at[t],
            send_sem=z_ss.at[t],
            recv_sem=z_rs.at[t],
            device_id=z_right,
            device_id_type=pl.DeviceIdType.MESH,
        )
        rdma.start()
        rdma.wait()
        row = base + c_recv * CHUNK
        out_ref[pl.ds(row, CHUNK), :] = out_ref[pl.ds(row, CHUNK), :] + z_comm[t]

    c0 = (mz + 1) % NZ
    HC = CHUNK // 2

    def z_hop(g):
        row = base + ((c0 - g) % NZ) * CHUNK
        return pltpu.make_async_remote_copy(
            src_ref=out_ref.at[pl.ds(row, CHUNK), :],
            dst_ref=out_ref.at[pl.ds(row, CHUNK), :],
            send_sem=z_ss.at[NZ - 1 + g],
            recv_sem=z_rs.at[NZ - 1 + g],
            device_id=z_right,
            device_id_type=pl.DeviceIdType.MESH,
        )

    def xy_pair(k, h):
        c = (c0 - k) % NZ
        row_cw = _slab_of_ring_pos((r - h) % N_SLABS) * SLAB + c * CHUNK
        row_ccw = _slab_of_ring_pos((r + h) % N_SLABS) * SLAB + c * CHUNK + HC
        idx = k * 6 + h * 2
        cw = pltpu.make_async_remote_copy(
            src_ref=out_ref.at[pl.ds(row_cw, HC), :],
            dst_ref=out_ref.at[pl.ds(row_cw, HC), :],
            send_sem=xy_ss.at[idx],
            recv_sem=xy_rs.at[idx],
            device_id=xy_right,
            device_id_type=pl.DeviceIdType.MESH,
        )
        ccw = pltpu.make_async_remote_copy(
            src_ref=out_ref.at[pl.ds(row_ccw, HC), :],
            dst_ref=out_ref.at[pl.ds(row_ccw, HC), :],
            send_sem=xy_ss.at[idx + 1],
            recv_sem=xy_rs.at[idx + 1],
            device_id=xy_left,
            device_id_type=pl.DeviceIdType.MESH,
        )
        return cw, ccw

    zh = [z_hop(g) for g in range(NZ - 1)]
    xy = {(k, h): xy_pair(k, h) for k in range(NZ) for h in range(N_SLABS - 1)}

    def start(k, h):
        xy[k, h][0].start()
        xy[k, h][1].start()

    def wait(k, h):
        xy[k, h][0].wait()
        xy[k, h][1].wait()

    zh[0].start()
    start(0, 0)
    wait(0, 0); start(0, 1)
    zh[0].wait()
    zh[1].start()
    start(1, 0)
    wait(0, 1); start(0, 2)
    wait(1, 0); start(1, 1)
    zh[1].wait()
    zh[2].start()
    start(2, 0)
    wait(0, 2)
    wait(1, 1); start(1, 2)
    wait(2, 0); start(2, 1)
    zh[2].wait()
    start(3, 0)
    wait(1, 2)
    wait(2, 1); start(2, 2)
    wait(3, 0); start(3, 1)
    wait(2, 2)
    wait(3, 1); start(3, 2)
    wait(3, 2)


def _allreduce_allgather(partial):
    return pl.pallas_call(
        _comm_body,
        out_shape=jax.ShapeDtypeStruct((M, N), jnp.float32),
        in_specs=[pl.BlockSpec(memory_space=pltpu.VMEM)],
        out_specs=pl.BlockSpec(memory_space=pltpu.VMEM),
        scratch_shapes=[
            pltpu.VMEM((NZ - 1, CHUNK, N), jnp.float32),
            pltpu.SemaphoreType.DMA((2 * (NZ - 1),)),
            pltpu.SemaphoreType.DMA((2 * (NZ - 1),)),
            pltpu.SemaphoreType.DMA((NZ * (N_SLABS - 1) * 2,)),
            pltpu.SemaphoreType.DMA((NZ * (N_SLABS - 1) * 2,)),
        ],
        compiler_params=_CompilerParams(collective_id=0),
    )(partial)


def kernel(dy, W):
    mx = lax.axis_index("x")
    my = lax.axis_index("y")
    s = 2 * mx + my
    partial = _local_gemm(dy, W, s)
    return _allreduce_allgather(partial)


# device time: 168380 ns/iter; 1.0016x vs baseline; 1.0016x over previous
import jax
import jax.numpy as jnp
from jax import lax
from jax.experimental import pallas as pl
from jax.experimental.pallas import tpu as pltpu

M = 2048
N = 2048
K = 8192
NX, NY, NZ = 2, 2, 4
N_SLABS = NX * NY
SLAB = M // N_SLABS
CHUNK = SLAB // NZ

BN = 512
BK = 2048

_CompilerParams = getattr(pltpu, "CompilerParams", None) or pltpu.TPUCompilerParams


def _gemm_body(s_ref, dy_ref, w_ref, out_ref, acc_ref):
    k = pl.program_id(1)

    @pl.when(k == 0)
    def _():
        acc_ref[...] = jnp.zeros_like(acc_ref)

    acc_ref[...] += lax.dot_general(
        dy_ref[...],
        w_ref[...],
        dimension_numbers=(((1,), (1,)), ((), ())),
        preferred_element_type=jnp.float32,
    )

    @pl.when(k == pl.num_programs(1) - 1)
    def _():
        out_ref[...] = acc_ref[...]


def _local_gemm(dy, w, s):
    grid_spec = pltpu.PrefetchScalarGridSpec(
        num_scalar_prefetch=1,
        grid=(N // BN, K // BK),
        in_specs=[
            pl.BlockSpec((SLAB, BK), lambda n, k, s_ref: (s_ref[0], k)),
            pl.BlockSpec((BN, BK), lambda n, k, s_ref: (n, k)),
        ],
        out_specs=pl.BlockSpec((SLAB, BN), lambda n, k, s_ref: (0, n)),
        scratch_shapes=[pltpu.VMEM((SLAB, BN), jnp.float32)],
    )
    return pl.pallas_call(
        _gemm_body,
        grid_spec=grid_spec,
        out_shape=jax.ShapeDtypeStruct((SLAB, N), jnp.float32),
        compiler_params=_CompilerParams(
            dimension_semantics=("parallel", "arbitrary"),
        ),
    )(jnp.asarray(s, jnp.int32).reshape(1), dy, w)


def _ring_pos(x, y):
    return 2 * x + jnp.bitwise_xor(x, y)


def _coords_of_ring_pos(p):
    x = p // 2
    y = ((p + 1) // 2) % 2
    return x, y


def _slab_of_ring_pos(p):
    x, y = _coords_of_ring_pos(p)
    return 2 * x + y


def _comm_body(partial_ref, out_ref, z_comm, z_ss, z_rs, xy_ss, xy_rs):
    mx = lax.axis_index("x")
    my = lax.axis_index("y")
    mz = lax.axis_index("z")
    s = 2 * mx + my
    r = _ring_pos(mx, my)
    base = s * SLAB

    z_right = (mx, my, (mz + 1) % NZ)
    z_left = (mx, my, (mz - 1) % NZ)
    xr, yr = _coords_of_ring_pos((r + 1) % N_SLABS)
    xl, yl = _coords_of_ring_pos((r - 1) % N_SLABS)
    xy_right = (xr, yr, mz)
    xy_left = (xl, yl, mz)

    bsem = pltpu.get_barrier_semaphore()
    for nbr in (z_left, z_right, xy_left, xy_right):
        pl.semaphore_signal(
            bsem, inc=1, device_id=nbr, device_id_type=pl.DeviceIdType.MESH
        )
    pl.semaphore_wait(bsem, 4)

    out_ref[pl.ds(base, SLAB), :] = partial_ref[...]

    for t in range(NZ - 1):
        c_send = (mz - t) % NZ
        c_recv = (mz - t - 1) % NZ
        rdma = pltpu.make_async_remote_copy(
            src_ref=out_ref.at[pl.ds(base + c_send * CHUNK, CHUNK), :],
            dst_ref=z_comm.at[t],
            send_sem=z_ss.at[t],
            recv_sem=z_rs.at[t],
            device_id=z_right,
            device_id_type=pl.DeviceIdType.MESH,
        )
        rdma.start()
        rdma.wait()
        row = base + c_recv * CHUNK
        out_ref[pl.ds(row, CHUNK), :] = out_ref[pl.ds(row, CHUNK), :] + z_comm[t]

    c0 = (mz + 1) % NZ
    HC = CHUNK // 2

    def z_hop(g):
        row = base + ((c0 - g) % NZ) * CHUNK
        return pltpu.make_async_remote_copy(
            src_ref=out_ref.at[pl.ds(row, CHUNK), :],
            dst_ref=out_ref.at[pl.ds(row, CHUNK), :],
            send_sem=z_ss.at[NZ - 1 + g],
            recv_sem=z_rs.at[NZ - 1 + g],
            device_id=z_right,
            device_id_type=pl.DeviceIdType.MESH,
        )

    def xy_pair(k, h):
        c = (c0 - k) % NZ
        row_cw = _slab_of_ring_pos((r - h) % N_SLABS) * SLAB + c * CHUNK
        row_ccw = _slab_of_ring_pos((r + h) % N_SLABS) * SLAB + c * CHUNK + HC
        idx = k * 6 + h * 2
        cw = pltpu.make_async_remote_copy(
            src_ref=out_ref.at[pl.ds(row_cw, HC), :],
            dst_ref=out_ref.at[pl.ds(row_cw, HC), :],
            send_sem=xy_ss.at[idx],
            recv_sem=xy_rs.at[idx],
            device_id=xy_right,
            device_id_type=pl.DeviceIdType.MESH,
        )
        ccw = pltpu.make_async_remote_copy(
            src_ref=out_ref.at[pl.ds(row_ccw, HC), :],
            dst_ref=out_ref.at[pl.ds(row_ccw, HC), :],
            send_sem=xy_ss.at[idx + 1],
            recv_sem=xy_rs.at[idx + 1],
            device_id=xy_left,
            device_id_type=pl.DeviceIdType.MESH,
        )
        return cw, ccw

    zh = [z_hop(g) for g in range(NZ - 1)]
    xy = {(k, h): xy_pair(k, h) for k in range(NZ) for h in range(N_SLABS - 1)}

    def start(k, h):
        xy[k, h][0].start()
        xy[k, h][1].start()

    def wait(k, h):
        xy[k, h][0].wait()
        xy[k, h][1].wait()

    zh[0].start()
    start(0, 0)
    wait(0, 0); start(0, 1)
    zh[0].wait()
    zh[1].start()
    start(1, 0)
    wait(0, 1); start(0, 2)
    wait(1, 0); start(1, 1)
    zh[1].wait()
    zh[2].start()
    start(2, 0)
    wait(0, 2)
    wait(1, 1); start(1, 2)
    wait(2, 0); start(2, 1)
    zh[2].wait()
    start(3, 0)
    wait(1, 2)
    wait(2, 1); start(2, 2)
    wait(3, 0); start(3, 1)
    wait(2, 2)
    wait(3, 1); start(3, 2)
    wait(3, 2)


def _allreduce_allgather(partial):
    return pl.pallas_call(
        _comm_body,
        out_shape=jax.ShapeDtypeStruct((M, N), jnp.float32),
        in_specs=[pl.BlockSpec(memory_space=pltpu.VMEM)],
        out_specs=pl.BlockSpec(memory_space=pltpu.VMEM),
        scratch_shapes=[
            pltpu.VMEM((NZ - 1, CHUNK, N), jnp.float32),
            pltpu.SemaphoreType.DMA((2 * (NZ - 1),)),
            pltpu.SemaphoreType.DMA((2 * (NZ - 1),)),
            pltpu.SemaphoreType.DMA((NZ * (N_SLABS - 1) * 2,)),
            pltpu.SemaphoreType.DMA((NZ * (N_SLABS - 1) * 2,)),
        ],
        compiler_params=_CompilerParams(collective_id=0),
    )(partial)


def kernel(dy, W):
    mx = lax.axis_index("x")
    my = lax.axis_index("y")
    s = 2 * mx + my
    partial = _local_gemm(dy, W, s)
    return _allreduce_allgather(partial)


# device time: 161990 ns/iter; 1.0411x vs baseline; 1.0394x over previous
import jax
import jax.numpy as jnp
from jax import lax
from jax.experimental import pallas as pl
from jax.experimental.pallas import tpu as pltpu

M = 2048
N = 2048
K = 8192
NX, NY, NZ = 2, 2, 4
N_SLABS = NX * NY
SLAB = M // N_SLABS
CHUNK = SLAB // NZ

BN = 512
BK = 2048

_CompilerParams = getattr(pltpu, "CompilerParams", None) or pltpu.TPUCompilerParams


def _gemm_body(s_ref, dy_ref, w_ref, out_ref, acc_ref):
    k = pl.program_id(0)
    n = pl.program_id(1)

    @pl.when(k == 0)
    def _():
        acc_ref[:, pl.ds(n * BN, BN)] = jnp.zeros((SLAB, BN), jnp.float32)

    acc_ref[:, pl.ds(n * BN, BN)] += lax.dot_general(
        dy_ref[...],
        w_ref[...],
        dimension_numbers=(((1,), (1,)), ((), ())),
        preferred_element_type=jnp.float32,
    )

    @pl.when(k == pl.num_programs(0) - 1)
    def _():
        out_ref[...] = acc_ref[:, pl.ds(n * BN, BN)]


def _local_gemm(dy, w, s):
    grid_spec = pltpu.PrefetchScalarGridSpec(
        num_scalar_prefetch=1,
        grid=(K // BK, N // BN),
        in_specs=[
            pl.BlockSpec((SLAB, BK), lambda k, n, s_ref: (s_ref[0], k)),
            pl.BlockSpec((BN, BK), lambda k, n, s_ref: (n, k)),
        ],
        out_specs=pl.BlockSpec((SLAB, BN), lambda k, n, s_ref: (0, n)),
        scratch_shapes=[pltpu.VMEM((SLAB, N), jnp.float32)],
    )
    return pl.pallas_call(
        _gemm_body,
        grid_spec=grid_spec,
        out_shape=jax.ShapeDtypeStruct((SLAB, N), jnp.float32),
        compiler_params=_CompilerParams(
            dimension_semantics=("arbitrary", "arbitrary"),
        ),
    )(jnp.asarray(s, jnp.int32).reshape(1), dy, w)


def _ring_pos(x, y):
    return 2 * x + jnp.bitwise_xor(x, y)


def _coords_of_ring_pos(p):
    x = p // 2
    y = ((p + 1) // 2) % 2
    return x, y


def _slab_of_ring_pos(p):
    x, y = _coords_of_ring_pos(p)
    return 2 * x + y


def _comm_body(partial_ref, out_ref, z_comm, z_ss, z_rs, xy_ss, xy_rs):
    mx = lax.axis_index("x")
    my = lax.axis_index("y")
    mz = lax.axis_index("z")
    s = 2 * mx + my
    r = _ring_pos(mx, my)
    base = s * SLAB

    z_right = (mx, my, (mz + 1) % NZ)
    z_left = (mx, my, (mz - 1) % NZ)
    xr, yr = _coords_of_ring_pos((r + 1) % N_SLABS)
    xl, yl = _coords_of_ring_pos((r - 1) % N_SLABS)
    xy_right = (xr, yr, mz)
    xy_left = (xl, yl, mz)

    bsem = pltpu.get_barrier_semaphore()
    for nbr in (z_left, z_right, xy_left, xy_right):
        pl.semaphore_signal(
            bsem, inc=1, device_id=nbr, device_id_type=pl.DeviceIdType.MESH
        )
    pl.semaphore_wait(bsem, 4)

    out_ref[pl.ds(base, SLAB), :] = partial_ref[...]

    for t in range(NZ - 1):
        c_send = (mz - t) % NZ
        c_recv = (mz - t - 1) % NZ
        rdma = pltpu.make_async_remote_copy(
            src_ref=out_ref.at[pl.ds(base + c_send * CHUNK, CHUNK), :],
            dst_ref=z_comm.at[t],
            send_sem=z_ss.at[t],
            recv_sem=z_rs.at[t],
            device_id=z_right,
            device_id_type=pl.DeviceIdType.MESH,
        )
        rdma.start()
        rdma.wait()
        row = base + c_recv * CHUNK
        out_ref[pl.ds(row, CHUNK), :] = out_ref[pl.ds(row, CHUNK), :] + z_comm[t]

    c0 = (mz + 1) % NZ
    HC = CHUNK // 2

    def z_hop(g):
        row = base + ((c0 - g) % NZ) * CHUNK
        return pltpu.make_async_remote_copy(
            src_ref=out_ref.at[pl.ds(row, CHUNK), :],
            dst_ref=out_ref.at[pl.ds(row, CHUNK), :],
            send_sem=z_ss.at[NZ - 1 + g],
            recv_sem=z_rs.at[NZ - 1 + g],
            device_id=z_right,
            device_id_type=pl.DeviceIdType.MESH,
        )

    def xy_pair(k, h):
        c = (c0 - k) % NZ
        row_cw = _slab_of_ring_pos((r - h) % N_SLABS) * SLAB + c * CHUNK
        row_ccw = _slab_of_ring_pos((r + h) % N_SLABS) * SLAB + c * CHUNK + HC
        idx = k * 6 + h * 2
        cw = pltpu.make_async_remote_copy(
            src_ref=out_ref.at[pl.ds(row_cw, HC), :],
            dst_ref=out_ref.at[pl.ds(row_cw, HC), :],
            send_sem=xy_ss.at[idx],
            recv_sem=xy_rs.at[idx],
            device_id=xy_right,
            device_id_type=pl.DeviceIdType.MESH,
        )
        ccw = pltpu.make_async_remote_copy(
            src_ref=out_ref.at[pl.ds(row_ccw, HC), :],
            dst_ref=out_ref.at[pl.ds(row_ccw, HC), :],
            send_sem=xy_ss.at[idx + 1],
            recv_sem=xy_rs.at[idx + 1],
            device_id=xy_left,
            device_id_type=pl.DeviceIdType.MESH,
        )
        return cw, ccw

    zh = [z_hop(g) for g in range(NZ - 1)]
    xy = {(k, h): xy_pair(k, h) for k in range(NZ) for h in range(N_SLABS - 1)}

    def start(k, h):
        xy[k, h][0].start()
        xy[k, h][1].start()

    def wait(k, h):
        xy[k, h][0].wait()
        xy[k, h][1].wait()

    zh[0].start()
    start(0, 0)
    wait(0, 0); start(0, 1)
    zh[0].wait()
    zh[1].start()
    start(1, 0)
    wait(0, 1); start(0, 2)
    wait(1, 0); start(1, 1)
    zh[1].wait()
    zh[2].start()
    start(2, 0)
    wait(0, 2)
    wait(1, 1); start(1, 2)
    wait(2, 0); start(2, 1)
    zh[2].wait()
    start(3, 0)
    wait(1, 2)
    wait(2, 1); start(2, 2)
    wait(3, 0); start(3, 1)
    wait(2, 2)
    wait(3, 1); start(3, 2)
    wait(3, 2)


def _allreduce_allgather(partial):
    return pl.pallas_call(
        _comm_body,
        out_shape=jax.ShapeDtypeStruct((M, N), jnp.float32),
        in_specs=[pl.BlockSpec(memory_space=pltpu.VMEM)],
        out_specs=pl.BlockSpec(memory_space=pltpu.VMEM),
        scratch_shapes=[
            pltpu.VMEM((NZ - 1, CHUNK, N), jnp.float32),
            pltpu.SemaphoreType.DMA((2 * (NZ - 1),)),
            pltpu.SemaphoreType.DMA((2 * (NZ - 1),)),
            pltpu.SemaphoreType.DMA((NZ * (N_SLABS - 1) * 2,)),
            pltpu.SemaphoreType.DMA((NZ * (N_SLABS - 1) * 2,)),
        ],
        compiler_params=_CompilerParams(collective_id=0),
    )(partial)


def kernel(dy, W):
    mx = lax.axis_index("x")
    my = lax.axis_index("y")
    s = 2 * mx + my
    partial = _local_gemm(dy, W, s)
    return _allreduce_allgather(partial)


# device time: 157705 ns/iter; 1.0694x vs baseline; 1.0272x over previous
import jax
import jax.numpy as jnp
from jax import lax
from jax.experimental import pallas as pl
from jax.experimental.pallas import tpu as pltpu

M = 2048
N = 2048
K = 8192
NX, NY, NZ = 2, 2, 4
N_SLABS = NX * NY
SLAB = M // N_SLABS
CHUNK = SLAB // NZ

BN = 512
BK = 2048

_CompilerParams = getattr(pltpu, "CompilerParams", None) or pltpu.TPUCompilerParams


NK = K // BK
NN = N // BN


def _gemm_rs_body(s_ref, dy_ref, w_ref, out_ref, acc_ref, comm_ref, ss, rs_):
    k = pl.program_id(0)
    n = pl.program_id(1)
    mx = lax.axis_index("x")
    my = lax.axis_index("y")
    mz = lax.axis_index("z")
    z_right = (mx, my, (mz + 1) % NZ)
    z_left = (mx, my, (mz - 1) % NZ)

    @pl.when(jnp.logical_and(k == 0, n == 0))
    def _():
        bsem = pltpu.get_barrier_semaphore()
        for nbr in (z_left, z_right):
            pl.semaphore_signal(
                bsem, inc=1, device_id=nbr, device_id_type=pl.DeviceIdType.MESH
            )
        pl.semaphore_wait(bsem, 2)

    @pl.when(k == 0)
    def _():
        acc_ref[:, pl.ds(n * BN, BN)] = jnp.zeros((SLAB, BN), jnp.float32)

    acc_ref[:, pl.ds(n * BN, BN)] += lax.dot_general(
        dy_ref[...],
        w_ref[...],
        dimension_numbers=(((1,), (1,)), ((), ())),
        preferred_element_type=jnp.float32,
    )

    def hop(j, t):
        return pltpu.make_async_remote_copy(
            src_ref=acc_ref.at[
                pl.ds(((mz - t) % NZ) * CHUNK, CHUNK), pl.ds(j * BN, BN)
            ],
            dst_ref=comm_ref.at[j * (NZ - 1) + t],
            send_sem=ss.at[j * (NZ - 1) + t],
            recv_sem=rs_.at[j * (NZ - 1) + t],
            device_id=z_right,
            device_id_type=pl.DeviceIdType.MESH,
        )

    def start(j, t):
        hop(j, t).start()

    def finish(j, t):
        hop(j, t).wait()
        row = ((mz - t - 1) % NZ) * CHUNK
        acc_ref[pl.ds(row, CHUNK), pl.ds(j * BN, BN)] += comm_ref[
            j * (NZ - 1) + t
        ]

    last_k = k == NK - 1

    @pl.when(jnp.logical_and(last_k, n == 0))
    def _():
        start(0, 0)

    @pl.when(jnp.logical_and(last_k, n == 1))
    def _():
        finish(0, 0); start(0, 1); start(1, 0)

    @pl.when(jnp.logical_and(last_k, n == 2))
    def _():
        finish(0, 1); start(0, 2)
        finish(1, 0); start(1, 1)
        start(2, 0)

    @pl.when(jnp.logical_and(last_k, n == 3))
    def _():
        finish(0, 2)
        finish(1, 1); start(1, 2)
        finish(2, 0); start(2, 1)
        start(3, 0)
        finish(1, 2)
        finish(2, 1); start(2, 2)
        finish(3, 0); start(3, 1)
        finish(2, 2)
        finish(3, 1); start(3, 2)
        finish(3, 2)
        out_ref[...] = acc_ref[pl.ds(((mz + 1) % NZ) * CHUNK, CHUNK), :]


def _local_gemm_rs(dy, w, s):
    grid_spec = pltpu.PrefetchScalarGridSpec(
        num_scalar_prefetch=1,
        grid=(NK, NN),
        in_specs=[
            pl.BlockSpec((SLAB, BK), lambda k, n, s_ref: (s_ref[0], k)),
            pl.BlockSpec((BN, BK), lambda k, n, s_ref: (n, k)),
        ],
        out_specs=pl.BlockSpec((CHUNK, N), lambda k, n, s_ref: (0, 0)),
        scratch_shapes=[
            pltpu.VMEM((SLAB, N), jnp.float32),
            pltpu.VMEM((NN * (NZ - 1), CHUNK, BN), jnp.float32),
            pltpu.SemaphoreType.DMA((NN * (NZ - 1),)),
            pltpu.SemaphoreType.DMA((NN * (NZ - 1),)),
        ],
    )
    return pl.pallas_call(
        _gemm_rs_body,
        grid_spec=grid_spec,
        out_shape=jax.ShapeDtypeStruct((CHUNK, N), jnp.float32),
        compiler_params=_CompilerParams(
            dimension_semantics=("arbitrary", "arbitrary"),
            collective_id=1,
        ),
    )(jnp.asarray(s, jnp.int32).reshape(1), dy, w)


def _ring_pos(x, y):
    return 2 * x + jnp.bitwise_xor(x, y)


def _coords_of_ring_pos(p):
    x = p // 2
    y = ((p + 1) // 2) % 2
    return x, y


def _slab_of_ring_pos(p):
    x, y = _coords_of_ring_pos(p)
    return 2 * x + y


def _comm_body(chunk_ref, out_ref, z_ss, z_rs, xy_ss, xy_rs):
    mx = lax.axis_index("x")
    my = lax.axis_index("y")
    mz = lax.axis_index("z")
    s = 2 * mx + my
    r = _ring_pos(mx, my)
    base = s * SLAB

    z_right = (mx, my, (mz + 1) % NZ)
    z_left = (mx, my, (mz - 1) % NZ)
    xr, yr = _coords_of_ring_pos((r + 1) % N_SLABS)
    xl, yl = _coords_of_ring_pos((r - 1) % N_SLABS)
    xy_right = (xr, yr, mz)
    xy_left = (xl, yl, mz)

    bsem = pltpu.get_barrier_semaphore()
    for nbr in (z_left, z_right, xy_left, xy_right):
        pl.semaphore_signal(
            bsem, inc=1, device_id=nbr, device_id_type=pl.DeviceIdType.MESH
        )
    pl.semaphore_wait(bsem, 4)

    c0_row = base + ((mz + 1) % NZ) * CHUNK
    out_ref[pl.ds(c0_row, CHUNK), :] = chunk_ref[...]

    c0 = (mz + 1) % NZ
    HC = CHUNK // 2

    def z_hop(g):
        row = base + ((c0 - g) % NZ) * CHUNK
        return pltpu.make_async_remote_copy(
            src_ref=out_ref.at[pl.ds(row, CHUNK), :],
            dst_ref=out_ref.at[pl.ds(row, CHUNK), :],
            send_sem=z_ss.at[g],
            recv_sem=z_rs.at[g],
            device_id=z_right,
            device_id_type=pl.DeviceIdType.MESH,
        )

    def xy_pair(k, h):
        c = (c0 - k) % NZ
        row_cw = _slab_of_ring_pos((r - h) % N_SLABS) * SLAB + c * CHUNK
        row_ccw = _slab_of_ring_pos((r + h) % N_SLABS) * SLAB + c * CHUNK + HC
        idx = k * 6 + h * 2
        cw = pltpu.make_async_remote_copy(
            src_ref=out_ref.at[pl.ds(row_cw, HC), :],
            dst_ref=out_ref.at[pl.ds(row_cw, HC), :],
            send_sem=xy_ss.at[idx],
            recv_sem=xy_rs.at[idx],
            device_id=xy_right,
            device_id_type=pl.DeviceIdType.MESH,
        )
        ccw = pltpu.make_async_remote_copy(
            src_ref=out_ref.at[pl.ds(row_ccw, HC), :],
            dst_ref=out_ref.at[pl.ds(row_ccw, HC), :],
            send_sem=xy_ss.at[idx + 1],
            recv_sem=xy_rs.at[idx + 1],
            device_id=xy_left,
            device_id_type=pl.DeviceIdType.MESH,
        )
        return cw, ccw

    zh = [z_hop(g) for g in range(NZ - 1)]
    xy = {(k, h): xy_pair(k, h) for k in range(NZ) for h in range(N_SLABS - 1)}

    def start(k, h):
        xy[k, h][0].start()
        xy[k, h][1].start()

    def wait(k, h):
        xy[k, h][0].wait()
        xy[k, h][1].wait()

    zh[0].start()
    start(0, 0)
    wait(0, 0); start(0, 1)
    zh[0].wait()
    zh[1].start()
    start(1, 0)
    wait(0, 1); start(0, 2)
    wait(1, 0); start(1, 1)
    zh[1].wait()
    zh[2].start()
    start(2, 0)
    wait(0, 2)
    wait(1, 1); start(1, 2)
    wait(2, 0); start(2, 1)
    zh[2].wait()
    start(3, 0)
    wait(1, 2)
    wait(2, 1); start(2, 2)
    wait(3, 0); start(3, 1)
    wait(2, 2)
    wait(3, 1); start(3, 2)
    wait(3, 2)


def _allreduce_allgather(chunk):
    return pl.pallas_call(
        _comm_body,
        out_shape=jax.ShapeDtypeStruct((M, N), jnp.float32),
        in_specs=[pl.BlockSpec(memory_space=pltpu.VMEM)],
        out_specs=pl.BlockSpec(memory_space=pltpu.VMEM),
        scratch_shapes=[
            pltpu.SemaphoreType.DMA((NZ - 1,)),
            pltpu.SemaphoreType.DMA((NZ - 1,)),
            pltpu.SemaphoreType.DMA((NZ * (N_SLABS - 1) * 2,)),
            pltpu.SemaphoreType.DMA((NZ * (N_SLABS - 1) * 2,)),
        ],
        compiler_params=_CompilerParams(collective_id=0),
    )(chunk)


def kernel(dy, W):
    mx = lax.axis_index("x")
    my = lax.axis_index("y")
    s = 2 * mx + my
    chunk = _local_gemm_rs(dy, W, s)
    return _allreduce_allgather(chunk)


# device time: 152299 ns/iter; 1.1073x vs baseline; 1.0355x over previous
import jax
import jax.numpy as jnp
from jax import lax
from jax.experimental import pallas as pl
from jax.experimental.pallas import tpu as pltpu

M = 2048
N = 2048
K = 8192
NX, NY, NZ = 2, 2, 4
N_SLABS = NX * NY
SLAB = M // N_SLABS
CHUNK = SLAB // NZ

BN = 512
BK = 2048

_CompilerParams = getattr(pltpu, "CompilerParams", None) or pltpu.TPUCompilerParams


NK = K // BK
NN = N // BN


def _gemm_rs_body(s_ref, dy_ref, w_ref, out_ref, acc_ref, red_ref, comm_ref, ss, rs_):
    n = pl.program_id(0)
    k = pl.program_id(1)
    mx = lax.axis_index("x")
    my = lax.axis_index("y")
    mz = lax.axis_index("z")
    z_right = (mx, my, (mz + 1) % NZ)
    z_left = (mx, my, (mz - 1) % NZ)

    @pl.when(jnp.logical_and(k == 0, n == 0))
    def _():
        bsem = pltpu.get_barrier_semaphore()
        for nbr in (z_left, z_right):
            pl.semaphore_signal(
                bsem, inc=1, device_id=nbr, device_id_type=pl.DeviceIdType.MESH
            )
        pl.semaphore_wait(bsem, 2)

    @pl.when(k == 0)
    def _():
        acc_ref[...] = jnp.zeros_like(acc_ref)

    acc_ref[...] += lax.dot_general(
        dy_ref[...],
        w_ref[...],
        dimension_numbers=(((1,), (1,)), ((), ())),
        preferred_element_type=jnp.float32,
    )

    def hop(j, t):
        return pltpu.make_async_remote_copy(
            src_ref=red_ref.at[
                pl.ds(((mz - t) % NZ) * CHUNK, CHUNK), pl.ds(j * BN, BN)
            ],
            dst_ref=comm_ref.at[j * (NZ - 1) + t],
            send_sem=ss.at[j * (NZ - 1) + t],
            recv_sem=rs_.at[j * (NZ - 1) + t],
            device_id=z_right,
            device_id_type=pl.DeviceIdType.MESH,
        )

    def start(j, t):
        hop(j, t).start()

    def finish(j, t):
        hop(j, t).wait()
        row = ((mz - t - 1) % NZ) * CHUNK
        red_ref[pl.ds(row, CHUNK), pl.ds(j * BN, BN)] += comm_ref[
            j * (NZ - 1) + t
        ]

    def stripe_done(j):
        red_ref[:, pl.ds(j * BN, BN)] = acc_ref[...]
        start(j, 0)

    def when(j, kk, *actions):
        def body():
            for a in actions:
                a()
        pl.when(jnp.logical_and(n == j, k == kk))(body)

    when(0, 3, lambda: stripe_done(0))
    when(1, 1, lambda: finish(0, 0), lambda: start(0, 1))
    when(1, 3, lambda: finish(0, 1), lambda: start(0, 2), lambda: stripe_done(1))
    when(2, 1, lambda: finish(0, 2), lambda: finish(1, 0), lambda: start(1, 1))
    when(2, 3, lambda: finish(1, 1), lambda: start(1, 2), lambda: stripe_done(2))
    when(3, 1, lambda: finish(1, 2), lambda: finish(2, 0), lambda: start(2, 1))

    @pl.when(jnp.logical_and(n == 3, k == 3))
    def _():
        finish(2, 1); start(2, 2)
        stripe_done(3)
        finish(2, 2)
        finish(3, 0); start(3, 1)
        finish(3, 1); start(3, 2)
        finish(3, 2)
        out_ref[...] = red_ref[pl.ds(((mz + 1) % NZ) * CHUNK, CHUNK), :]


def _local_gemm_rs(dy, w, s):
    grid_spec = pltpu.PrefetchScalarGridSpec(
        num_scalar_prefetch=1,
        grid=(NN, NK),
        in_specs=[
            pl.BlockSpec((SLAB, BK), lambda n, k, s_ref: (s_ref[0], k)),
            pl.BlockSpec((BN, BK), lambda n, k, s_ref: (n, k)),
        ],
        out_specs=pl.BlockSpec((CHUNK, N), lambda n, k, s_ref: (0, 0)),
        scratch_shapes=[
            pltpu.VMEM((SLAB, BN), jnp.float32),
            pltpu.VMEM((SLAB, N), jnp.float32),
            pltpu.VMEM((NN * (NZ - 1), CHUNK, BN), jnp.float32),
            pltpu.SemaphoreType.DMA((NN * (NZ - 1),)),
            pltpu.SemaphoreType.DMA((NN * (NZ - 1),)),
        ],
    )
    return pl.pallas_call(
        _gemm_rs_body,
        grid_spec=grid_spec,
        out_shape=jax.ShapeDtypeStruct((CHUNK, N), jnp.float32),
        compiler_params=_CompilerParams(
            dimension_semantics=("arbitrary", "arbitrary"),
            collective_id=1,
        ),
    )(jnp.asarray(s, jnp.int32).reshape(1), dy, w)


def _ring_pos(x, y):
    return 2 * x + jnp.bitwise_xor(x, y)


def _coords_of_ring_pos(p):
    x = p // 2
    y = ((p + 1) // 2) % 2
    return x, y


def _slab_of_ring_pos(p):
    x, y = _coords_of_ring_pos(p)
    return 2 * x + y


def _comm_body(chunk_ref, out_ref, z_ss, z_rs, xy_ss, xy_rs):
    mx = lax.axis_index("x")
    my = lax.axis_index("y")
    mz = lax.axis_index("z")
    s = 2 * mx + my
    r = _ring_pos(mx, my)
    base = s * SLAB

    z_right = (mx, my, (mz + 1) % NZ)
    z_left = (mx, my, (mz - 1) % NZ)
    xr, yr = _coords_of_ring_pos((r + 1) % N_SLABS)
    xl, yl = _coords_of_ring_pos((r - 1) % N_SLABS)
    xy_right = (xr, yr, mz)
    xy_left = (xl, yl, mz)

    bsem = pltpu.get_barrier_semaphore()
    for nbr in (z_left, z_right, xy_left, xy_right):
        pl.semaphore_signal(
            bsem, inc=1, device_id=nbr, device_id_type=pl.DeviceIdType.MESH
        )
    pl.semaphore_wait(bsem, 4)

    c0_row = base + ((mz + 1) % NZ) * CHUNK
    out_ref[pl.ds(c0_row, CHUNK), :] = chunk_ref[...]

    c0 = (mz + 1) % NZ
    HC = CHUNK // 2

    def z_hop(g):
        row = base + ((c0 - g) % NZ) * CHUNK
        return pltpu.make_async_remote_copy(
            src_ref=out_ref.at[pl.ds(row, CHUNK), :],
            dst_ref=out_ref.at[pl.ds(row, CHUNK), :],
            send_sem=z_ss.at[g],
            recv_sem=z_rs.at[g],
            device_id=z_right,
            device_id_type=pl.DeviceIdType.MESH,
        )

    def xy_pair(k, h):
        c = (c0 - k) % NZ
        row_cw = _slab_of_ring_pos((r - h) % N_SLABS) * SLAB + c * CHUNK
        row_ccw = _slab_of_ring_pos((r + h) % N_SLABS) * SLAB + c * CHUNK + HC
        idx = k * 6 + h * 2
        cw = pltpu.make_async_remote_copy(
            src_ref=out_ref.at[pl.ds(row_cw, HC), :],
            dst_ref=out_ref.at[pl.ds(row_cw, HC), :],
            send_sem=xy_ss.at[idx],
            recv_sem=xy_rs.at[idx],
            device_id=xy_right,
            device_id_type=pl.DeviceIdType.MESH,
        )
        ccw = pltpu.make_async_remote_copy(
            src_ref=out_ref.at[pl.ds(row_ccw, HC), :],
            dst_ref=out_ref.at[pl.ds(row_ccw, HC), :],
            send_sem=xy_ss.at[idx + 1],
            recv_sem=xy_rs.at[idx + 1],
            device_id=xy_left,
            device_id_type=pl.DeviceIdType.MESH,
        )
        return cw, ccw

    zh = [z_hop(g) for g in range(NZ - 1)]
    xy = {(k, h): xy_pair(k, h) for k in range(NZ) for h in range(N_SLABS - 1)}

    def start(k, h):
        xy[k, h][0].start()
        xy[k, h][1].start()

    def wait(k, h):
        xy[k, h][0].wait()
        xy[k, h][1].wait()

    zh[0].start()
    start(0, 0)
    wait(0, 0); start(0, 1)
    zh[0].wait()
    zh[1].start()
    start(1, 0)
    wait(0, 1); start(0, 2)
    wait(1, 0); start(1, 1)
    zh[1].wait()
    zh[2].start()
    start(2, 0)
    wait(0, 2)
    wait(1, 1); start(1, 2)
    wait(2, 0); start(2, 1)
    zh[2].wait()
    start(3, 0)
    wait(1, 2)
    wait(2, 1); start(2, 2)
    wait(3, 0); start(3, 1)
    wait(2, 2)
    wait(3, 1); start(3, 2)
    wait(3, 2)


def _allreduce_allgather(chunk):
    return pl.pallas_call(
        _comm_body,
        out_shape=jax.ShapeDtypeStruct((M, N), jnp.float32),
        in_specs=[pl.BlockSpec(memory_space=pltpu.VMEM)],
        out_specs=pl.BlockSpec(memory_space=pltpu.VMEM),
        scratch_shapes=[
            pltpu.SemaphoreType.DMA((NZ - 1,)),
            pltpu.SemaphoreType.DMA((NZ - 1,)),
            pltpu.SemaphoreType.DMA((NZ * (N_SLABS - 1) * 2,)),
            pltpu.SemaphoreType.DMA((NZ * (N_SLABS - 1) * 2,)),
        ],
        compiler_params=_CompilerParams(collective_id=0),
    )(chunk)


def kernel(dy, W):
    mx = lax.axis_index("x")
    my = lax.axis_index("y")
    s = 2 * mx + my
    chunk = _local_gemm_rs(dy, W, s)
    return _allreduce_allgather(chunk)
